# baseline (device time: 859095 ns/iter reference)
import jax
import jax.numpy as jnp
from jax import lax
from jax.experimental import pallas as pl
from jax.experimental.pallas import tpu as pltpu

T_CHUNK = 128
LOC_SLOTS = 3
RECV_SLOTS = 2
W_TILE = 512


def kernel(x, W):
    t, d = x.shape
    _, v = W.shape
    n = t // T_CHUNK
    C = v // W_TILE
    G = n * C

    def body(
        x_ref, w_ref, out_ref,
        wc, xc, loc, recv, osl,
        wsems, xsems, send_sems, recv_sems, out_sem, credit_sem,
    ):
        my_x = lax.axis_index("x")
        my_y = lax.axis_index("y")
        my_z = lax.axis_index("z")
        partner = (1 - my_x, my_y, my_z)
        T = T_CHUNK

        def rem(a, k):
            return a % k if isinstance(a, int) else lax.rem(a, k)

        def wdma(g):
            return pltpu.make_async_copy(
                w_ref.at[:, pl.ds(rem(g, C) * W_TILE, W_TILE)],
                wc.at[rem(g, 2)],
                wsems.at[rem(g, 2)],
            )

        def xload(j):
            return pltpu.make_async_copy(
                x_ref.at[pl.ds(j * T, T), :],
                xc.at[rem(j, 2)],
                xsems.at[rem(j, 2)],
            )

        def rdma(j):
            return pltpu.make_async_remote_copy(
                src_ref=loc.at[rem(j, LOC_SLOTS)],
                dst_ref=recv.at[rem(j, RECV_SLOTS)],
                send_sem=send_sems.at[rem(j, LOC_SLOTS)],
                recv_sem=recv_sems.at[rem(j, RECV_SLOTS)],
                device_id=partner,
                device_id_type=pl.DeviceIdType.MESH,
            )

        def out_dma(j):
            return pltpu.make_async_copy(
                osl, out_ref.at[pl.ds(j * T, T), :], out_sem
            )

        def gemm(j):
            jj = rem(j, LOC_SLOTS)
            xs = rem(j, 2)
            base = j * C

            def tile(c, _):
                g = base + c
                wdma(g).wait()

                @pl.when(g + 2 < G)
                def _():
                    wdma(g + 2).start()

                loc[jj, :, pl.ds(c * W_TILE, W_TILE)] = jnp.dot(
                    xc[xs], wc[rem(g, 2)],
                    preferred_element_type=jnp.float32,
                )
                return 0

            lax.fori_loop(0, C, tile, 0)

        xload(0).start()
        barrier = pltpu.get_barrier_semaphore()
        pl.semaphore_signal(
            barrier, inc=1, device_id=partner,
            device_id_type=pl.DeviceIdType.MESH,
        )
        pl.semaphore_wait(barrier, 1)
        wdma(0).start()
        wdma(1).start()
        xload(0).wait()
        gemm(0)
        xload(1).start()
        rdma(0).start()

        def step(i, _):
            @pl.when(i + 1 < n)
            def _():
                xload(i + 1).wait()

                @pl.when(i + 1 >= LOC_SLOTS)
                def _():
                    rdma(i + 1 - LOC_SLOTS).wait_send()

                gemm(i + 1)

                @pl.when(i + 2 < n)
                def _():
                    xload(i + 2).start()

                @pl.when(i + 1 >= RECV_SLOTS)
                def _():
                    pl.semaphore_wait(credit_sem, 1)

                rdma(i + 1).start()

            rdma(i).wait_recv()

            @pl.when(i >= 1)
            def _():
                out_dma(i - 1).wait()

            lo = loc[rem(i, LOC_SLOTS)]
            rm = recv[rem(i, RECV_SLOTS)]
            m = jnp.maximum(
                jnp.max(lo, axis=-1, keepdims=True),
                jnp.max(rm, axis=-1, keepdims=True),
            )
            el = jnp.exp(lo - m)
            er = jnp.exp(rm - m)
            den = (
                jnp.sum(el, axis=-1, keepdims=True)
                + jnp.sum(er, axis=-1, keepdims=True)
            )
            osl[:, pl.ds(my_x * v, v)] = el / den
            osl[:, pl.ds((1 - my_x) * v, v)] = er / den
            pl.semaphore_signal(
                credit_sem, inc=1, device_id=partner,
                device_id_type=pl.DeviceIdType.MESH,
            )
            out_dma(i).start()
            return 0

        lax.fori_loop(0, n, step, 0)

        for j in range(n - LOC_SLOTS, n):
            rdma(j).wait_send()
        out_dma(n - 1).wait()
        pl.semaphore_wait(credit_sem, RECV_SLOTS)

    return pl.pallas_call(
        body,
        in_specs=[
            pl.BlockSpec(memory_space=pl.ANY),
            pl.BlockSpec(memory_space=pl.ANY),
        ],
        out_specs=pl.BlockSpec(memory_space=pl.ANY),
        out_shape=jax.ShapeDtypeStruct((t, 2 * v), jnp.float32),
        scratch_shapes=[
            pltpu.VMEM((2, d, W_TILE), jnp.float32),
            pltpu.VMEM((2, T_CHUNK, d), jnp.float32),
            pltpu.VMEM((LOC_SLOTS, T_CHUNK, v), jnp.float32),
            pltpu.VMEM((RECV_SLOTS, T_CHUNK, v), jnp.float32),
            pltpu.VMEM((T_CHUNK, 2 * v), jnp.float32),
            pltpu.SemaphoreType.DMA((2,)),
            pltpu.SemaphoreType.DMA((2,)),
            pltpu.SemaphoreType.DMA((LOC_SLOTS,)),
            pltpu.SemaphoreType.DMA((RECV_SLOTS,)),
            pltpu.SemaphoreType.DMA,
            pltpu.SemaphoreType.REGULAR,
        ],
        compiler_params=pltpu.CompilerParams(
            collective_id=0, vmem_limit_bytes=63 * 1024 * 1024
        ),
    )(x, W)


# device time: 856689 ns/iter; 1.0028x vs baseline; 1.0028x over previous
import jax
import jax.numpy as jnp
from jax import lax
from jax.experimental import pallas as pl
from jax.experimental.pallas import tpu as pltpu

T_CHUNK = 128
PAIR = 2 * T_CHUNK
RECV_SLOTS = 2
W_TILE = 256


def kernel(x, W):
    t, d = x.shape
    _, v = W.shape
    n = t // T_CHUNK
    npairs = n // 2
    C = v // W_TILE
    G = npairs * C

    def body(
        x_ref, w_ref, out_ref,
        wc, xp, loc, recv, osl,
        wsems, xsems, send_sems, recv_sems, out_sem, credit_sem,
    ):
        my_x = lax.axis_index("x")
        my_y = lax.axis_index("y")
        my_z = lax.axis_index("z")
        partner = (1 - my_x, my_y, my_z)
        T = T_CHUNK

        def rem(a, k):
            return a % k if isinstance(a, int) else lax.rem(a, k)

        def div(a, k):
            return a // k if isinstance(a, int) else lax.div(a, k)

        def wdma(g):
            return pltpu.make_async_copy(
                w_ref.at[:, pl.ds(rem(g, C) * W_TILE, W_TILE)],
                wc.at[rem(g, 2)],
                wsems.at[rem(g, 2)],
            )

        def xpl(q):
            return pltpu.make_async_copy(
                x_ref.at[pl.ds(q * PAIR, PAIR), :],
                xp.at[rem(q, 2)],
                xsems.at[rem(q, 2)],
            )

        def rdma(j):
            return pltpu.make_async_remote_copy(
                src_ref=loc.at[rem(div(j, 2), 2), pl.ds(rem(j, 2) * T, T)],
                dst_ref=recv.at[rem(j, RECV_SLOTS)],
                send_sem=send_sems.at[rem(j, 2)],
                recv_sem=recv_sems.at[rem(j, RECV_SLOTS)],
                device_id=partner,
                device_id_type=pl.DeviceIdType.MESH,
            )

        def out_dma(j):
            return pltpu.make_async_copy(
                osl, out_ref.at[pl.ds(j * T, T), :], out_sem
            )

        def gemm_pair(q):
            ps = rem(q, 2)
            base = q * C

            def tile(c, _):
                g = base + c
                wdma(g).wait()

                @pl.when(g + 2 < G)
                def _():
                    wdma(g + 2).start()

                loc[ps, :, pl.ds(c * W_TILE, W_TILE)] = jnp.dot(
                    xp[rem(q, 2)], wc[rem(g, 2)],
                    preferred_element_type=jnp.float32,
                )
                return 0

            lax.fori_loop(0, C, tile, 0)

        xpl(0).start()
        barrier = pltpu.get_barrier_semaphore()
        pl.semaphore_signal(
            barrier, inc=1, device_id=partner,
            device_id_type=pl.DeviceIdType.MESH,
        )
        pl.semaphore_wait(barrier, 1)
        wdma(0).start()
        wdma(1).start()
        xpl(0).wait()
        gemm_pair(0)
        xpl(1).start()
        rdma(0).start()

        def step(i, _):
            @pl.when(i + 1 < n)
            def _():
                @pl.when(i + 1 >= 2)
                def _():
                    rdma(i - 1).wait_send()

                @pl.when(i + 1 >= RECV_SLOTS)
                def _():
                    pl.semaphore_wait(credit_sem, 1)

                rdma(i + 1).start()

            @pl.when((rem(i, 2) == 0) & (i + 2 < n))
            def _():
                q = div(i, 2) + 1
                xpl(q).wait()
                gemm_pair(q)

                @pl.when(q + 1 < npairs)
                def _():
                    xpl(q + 1).start()

            rdma(i).wait_recv()

            @pl.when(i >= 1)
            def _():
                out_dma(i - 1).wait()

            lo = loc[rem(div(i, 2), 2), pl.ds(rem(i, 2) * T, T)]
            rm = recv[rem(i, RECV_SLOTS)]
            m = jnp.maximum(
                jnp.max(lo, axis=-1, keepdims=True),
                jnp.max(rm, axis=-1, keepdims=True),
            )
            el = jnp.exp(lo - m)
            er = jnp.exp(rm - m)
            den = (
                jnp.sum(el, axis=-1, keepdims=True)
                + jnp.sum(er, axis=-1, keepdims=True)
            )
            osl[:, pl.ds(my_x * v, v)] = el / den
            osl[:, pl.ds((1 - my_x) * v, v)] = er / den
            pl.semaphore_signal(
                credit_sem, inc=1, device_id=partner,
                device_id_type=pl.DeviceIdType.MESH,
            )
            out_dma(i).start()
            return 0

        lax.fori_loop(0, n, step, 0)

        rdma(n - 2).wait_send()
        rdma(n - 1).wait_send()
        out_dma(n - 1).wait()
        pl.semaphore_wait(credit_sem, RECV_SLOTS)

    return pl.pallas_call(
        body,
        in_specs=[
            pl.BlockSpec(memory_space=pl.ANY),
            pl.BlockSpec(memory_space=pl.ANY),
        ],
        out_specs=pl.BlockSpec(memory_space=pl.ANY),
        out_shape=jax.ShapeDtypeStruct((t, 2 * v), jnp.float32),
        scratch_shapes=[
            pltpu.VMEM((2, d, W_TILE), jnp.float32),
            pltpu.VMEM((2, PAIR, d), jnp.float32),
            pltpu.VMEM((2, PAIR, v), jnp.float32),
            pltpu.VMEM((RECV_SLOTS, T_CHUNK, v), jnp.float32),
            pltpu.VMEM((T_CHUNK, 2 * v), jnp.float32),
            pltpu.SemaphoreType.DMA((2,)),
            pltpu.SemaphoreType.DMA((2,)),
            pltpu.SemaphoreType.DMA((2,)),
            pltpu.SemaphoreType.DMA((RECV_SLOTS,)),
            pltpu.SemaphoreType.DMA,
            pltpu.SemaphoreType.REGULAR,
        ],
        compiler_params=pltpu.CompilerParams(
            collective_id=0, vmem_limit_bytes=63 * 1024 * 1024
        ),
    )(x, W)
